# Initial kernel scaffold; baseline (speedup 1.0000x reference)
import jax
import jax.numpy as jnp
from jax.experimental import pallas as pl


def kernel(users, items, user_emb, item_emb, adj_rows, adj_cols, adj_vals):
    raise NotImplementedError("write your pallas kernel here")


# speedup vs baseline: 16.9265x; 16.9265x over previous
import functools

import jax
import jax.numpy as jnp
from jax import lax
from jax.experimental import pallas as pl
from jax.experimental.pallas import tpu as pltpu
from jax.experimental.pallas import tpu_sc as plsc

USERS = 50000
ITEMS = 50000
N = 100000
N_PAD = 100096
EMB = 32
HALF = 16
E = 1600000
SUB = 128
K = 10
CHUNK = K * SUB
NCH = E // CHUNK
NCORE = 2
NSUB = 16
TRIPS = -(-NCH // NSUB)
ROWS_PER_TILE = N_PAD // NSUB
BATCH = 4096
B2 = 2 * BATCH
BPT = B2 // NSUB

_PARAMS = pltpu.CompilerParams(use_tc_tiling_on_sc=False)

_MESH = plsc.VectorSubcoreMesh(
    core_axis_name="c", subcore_axis_name="s", num_cores=NCORE,
    num_subcores=NSUB)


def _spmm_body(ego, cols2, rows3, vals2, out,
               colbuf, rowbuf, valbuf, gth, acc, sem_e, sem_g, sem_s):
    c = lax.axis_index("c")
    tid = lax.axis_index("s")
    base_col = c * N_PAD

    def zbody(i, carry):
        gth[i, :] = jnp.zeros((HALF,), jnp.float32)
        return carry
    lax.fori_loop(0, CHUNK, zbody, None, unroll=8)
    row0 = tid * ROWS_PER_TILE
    for q in range(ROWS_PER_TILE // CHUNK):
        pltpu.sync_copy(gth, acc.at[pl.ds(row0 + q * CHUNK, CHUNK)])
    tail = ROWS_PER_TILE % CHUNK
    if tail:
        pltpu.sync_copy(gth.at[pl.ds(0, tail)],
                        acc.at[pl.ds(row0 + ROWS_PER_TILE - tail, tail)])
    plsc.subcore_barrier()

    def chunk_body(t, carry):
        i = t * NSUB + tid

        @pl.when(i < NCH)
        def _():
            d1 = pltpu.async_copy(cols2.at[i], colbuf, sem_e)
            d2 = pltpu.async_copy(rows3.at[i], rowbuf, sem_e)
            d3 = pltpu.async_copy(vals2.at[i], valbuf, sem_e)
            d1.wait()
            d2.wait()
            d3.wait()

            def adj(j, carry2):
                colbuf[pl.ds(j * 16, 16)] = colbuf[pl.ds(j * 16, 16)] + base_col
                return carry2
            lax.fori_loop(0, CHUNK // 16, adj, None, unroll=8)

            gds = [pltpu.async_copy(ego.at[colbuf.at[pl.ds(j * SUB, SUB)]],
                                    gth.at[pl.ds(j * SUB, SUB)], sem_g)
                   for j in range(K)]
            for d in gds:
                d.wait()

            def scale(g, carry2):
                v = valbuf[pl.ds(g * 16, 16)]
                base = g * 16
                for l in range(16):
                    gth[base + l, :] = gth[base + l, :] * v[l]
                return carry2
            lax.fori_loop(0, CHUNK // 16, scale, None, unroll=2)

            sds = [pltpu.async_copy(gth.at[pl.ds(j * SUB, SUB)],
                                    acc.at[rowbuf.at[j]], sem_s, add=True)
                   for j in range(K)]
            for d in sds:
                d.wait()
        return carry
    lax.fori_loop(0, TRIPS, chunk_body, None)

    plsc.subcore_barrier()
    pltpu.sync_copy(acc.at[pl.ds(row0, ROWS_PER_TILE)],
                    out.at[pl.ds(base_col + row0, ROWS_PER_TILE)])


_spmm = functools.partial(
    pl.kernel,
    out_type=jax.ShapeDtypeStruct((NCORE * N_PAD, HALF), jnp.float32),
    mesh=_MESH,
    compiler_params=_PARAMS,
    scratch_types=[
        pltpu.VMEM((CHUNK,), jnp.int32),
        pltpu.VMEM((K, SUB), jnp.int32),
        pltpu.VMEM((CHUNK,), jnp.float32),
        pltpu.VMEM((CHUNK, HALF), jnp.float32),
        pltpu.VMEM_SHARED((N_PAD, HALF), jnp.float32),
        pltpu.SemaphoreType.DMA,
        pltpu.SemaphoreType.DMA,
        pltpu.SemaphoreType.DMA,
    ],
)(_spmm_body)


def _final_body(e0, e1, e2, e3, nid3, out, nbuf, g0, g1, g2, g3, ob, sem):
    c = lax.axis_index("c")
    tid = lax.axis_index("s")
    pltpu.sync_copy(nid3.at[tid], nbuf)

    def adj(j, carry):
        nbuf[pl.ds(j * 16, 16)] = nbuf[pl.ds(j * 16, 16)] + c * N_PAD
        return carry
    lax.fori_loop(0, BPT // 16, adj, None, unroll=8)

    descs = []
    for tbl, g in ((e0, g0), (e1, g1), (e2, g2), (e3, g3)):
        for q in range(BPT // SUB):
            descs.append(
                pltpu.async_copy(tbl.at[nbuf.at[pl.ds(q * SUB, SUB)]],
                                 g.at[pl.ds(q * SUB, SUB)], sem))
    for d in descs:
        d.wait()

    def mean(e, carry):
        ob[e, :] = (g0[e, :] + g1[e, :] + g2[e, :] + g3[e, :]) * 0.25
        return carry
    lax.fori_loop(0, BPT, mean, None, unroll=8)

    pltpu.sync_copy(ob, out.at[pl.ds(c * B2 + tid * BPT, BPT)])


_final = functools.partial(
    pl.kernel,
    out_type=jax.ShapeDtypeStruct((NCORE * B2, HALF), jnp.float32),
    mesh=_MESH,
    compiler_params=_PARAMS,
    scratch_types=[
        pltpu.VMEM((BPT,), jnp.int32),
        pltpu.VMEM((BPT, HALF), jnp.float32),
        pltpu.VMEM((BPT, HALF), jnp.float32),
        pltpu.VMEM((BPT, HALF), jnp.float32),
        pltpu.VMEM((BPT, HALF), jnp.float32),
        pltpu.VMEM((BPT, HALF), jnp.float32),
        pltpu.SemaphoreType.DMA,
    ],
)(_final_body)


def kernel(users, items, user_emb, item_emb, adj_rows, adj_cols, adj_vals):
    ego0 = jnp.concatenate([user_emb, item_emb], axis=0)
    ego0 = jnp.pad(ego0, ((0, N_PAD - N), (0, 0)))
    ego0 = ego0.reshape(N_PAD, NCORE, HALF).transpose(1, 0, 2).reshape(NCORE * N_PAD, HALF)
    cols2 = adj_cols.reshape(NCH, CHUNK)
    rows3 = adj_rows.reshape(NCH, K, SUB)
    vals2 = adj_vals.reshape(NCH, CHUNK)

    e1 = _spmm(ego0, cols2, rows3, vals2)
    e2 = _spmm(e1, cols2, rows3, vals2)
    e3 = _spmm(e2, cols2, rows3, vals2)

    nid = jnp.concatenate(
        [users.astype(jnp.int32), items.astype(jnp.int32) + USERS])
    nid3 = nid.reshape(NSUB, BPT)
    outf = _final(ego0, e1, e2, e3, nid3)

    o = outf.reshape(NCORE, B2, HALF).transpose(1, 0, 2).reshape(B2, EMB)
    return (o[:BATCH], o[BATCH:])


# speedup vs baseline: 19.3143x; 1.1411x over previous
import functools

import jax
import jax.numpy as jnp
from jax import lax
from jax.experimental import pallas as pl
from jax.experimental.pallas import tpu as pltpu
from jax.experimental.pallas import tpu_sc as plsc

USERS = 50000
ITEMS = 50000
N = 100000
N_PAD = 100096
EMB = 32
HALF = 16
E = 1600000
SUB = 128
K = 5
CHUNK = K * SUB
NCH = E // CHUNK
NCORE = 2
NSUB = 16
TRIPS = -(-NCH // NSUB)
T2 = (TRIPS + 2) // 2
ROWS_PER_TILE = N_PAD // NSUB
BATCH = 4096
B2 = 2 * BATCH
BPT = B2 // NSUB

_PARAMS = pltpu.CompilerParams(use_tc_tiling_on_sc=False)

_MESH = plsc.VectorSubcoreMesh(
    core_axis_name="c", subcore_axis_name="s", num_cores=NCORE,
    num_subcores=NSUB)


def _spmm_body(ego, cols_both, rows3, vals2, out,
               cbA, cbB, rbA, rbB, vbA, vbB, gA, gB, acc,
               semA, semB, sem_gA, sem_gB, sem_sA, sem_sB, sem_z):
    c = lax.axis_index("c")
    tid = lax.axis_index("s")

    bufA = (cbA, rbA, vbA, gA, semA, sem_gA, sem_sA)
    bufB = (cbB, rbB, vbB, gB, semB, sem_gB, sem_sB)

    def stage_edges(i, buf):
        cb, rb, vb, sem = buf[0], buf[1], buf[2], buf[4]
        d1 = pltpu.async_copy(cols_both.at[c * NCH + i], cb, sem)
        d2 = pltpu.async_copy(rows3.at[i], rb, sem)
        d3 = pltpu.async_copy(vals2.at[i], vb, sem)
        d1.wait()
        d2.wait()
        d3.wait()

    def fire_gathers(buf):
        cb, g, sem_g = buf[0], buf[3], buf[5]
        for j in range(K):
            pltpu.async_copy(ego.at[cb.at[pl.ds(j * SUB, SUB)]],
                             g.at[pl.ds(j * SUB, SUB)], sem_g)

    def wait_gathers(buf):
        cb, g, sem_g = buf[0], buf[3], buf[5]
        for j in range(K):
            pltpu.make_async_copy(ego.at[cb.at[pl.ds(j * SUB, SUB)]],
                                  g.at[pl.ds(j * SUB, SUB)], sem_g).wait()

    def fire_scatters(buf):
        rb, g, sem_s = buf[1], buf[3], buf[6]
        for j in range(K):
            pltpu.async_copy(g.at[pl.ds(j * SUB, SUB)],
                             acc.at[rb.at[j]], sem_s, add=True)

    def wait_scatters(buf):
        rb, g, sem_s = buf[1], buf[3], buf[6]
        for j in range(K):
            pltpu.make_async_copy(g.at[pl.ds(j * SUB, SUB)],
                                  acc.at[rb.at[j]], sem_s).wait()

    def scale(buf):
        vb, g = buf[2], buf[3]

        def body(gi, carry):
            v = vb[pl.ds(gi * 16, 16)]
            base = gi * 16
            for l in range(16):
                g[base + l, :] = g[base + l, :] * v[l]
            return carry
        lax.fori_loop(0, CHUNK // 16, body, None, unroll=2)

    stage_edges(tid, bufA)
    fire_gathers(bufA)

    def zbody(i, carry):
        gB[i, :] = jnp.zeros((HALF,), jnp.float32)
        return carry
    lax.fori_loop(0, CHUNK, zbody, None, unroll=8)
    row0 = tid * ROWS_PER_TILE
    zds = []
    for q in range(ROWS_PER_TILE // CHUNK):
        zds.append(pltpu.async_copy(
            gB, acc.at[pl.ds(row0 + q * CHUNK, CHUNK)], sem_z))
    tail = ROWS_PER_TILE % CHUNK
    if tail:
        zds.append(pltpu.async_copy(
            gB.at[pl.ds(0, tail)],
            acc.at[pl.ds(row0 + ROWS_PER_TILE - tail, tail)], sem_z))
    for d in zds:
        d.wait()
    plsc.subcore_barrier()

    def half(t, cur, nxt):
        i_cur = t * NSUB + tid
        i_prev = i_cur - NSUB
        i_next = i_cur + NSUB

        @pl.when(i_cur < NCH)
        def _():
            wait_gathers(cur)

        @pl.when((t >= 1) & (i_prev < NCH))
        def _():
            wait_scatters(nxt)

        @pl.when(i_next < NCH)
        def _():
            stage_edges(i_next, nxt)
            fire_gathers(nxt)

        @pl.when(i_cur < NCH)
        def _():
            scale(cur)
            fire_scatters(cur)

    def pipe(t2, carry):
        half(2 * t2, bufA, bufB)
        half(2 * t2 + 1, bufB, bufA)
        return carry
    lax.fori_loop(0, T2, pipe, None)

    plsc.subcore_barrier()
    pltpu.sync_copy(acc.at[pl.ds(row0, ROWS_PER_TILE)],
                    out.at[pl.ds(c * N_PAD + row0, ROWS_PER_TILE)])


_spmm = functools.partial(
    pl.kernel,
    out_type=jax.ShapeDtypeStruct((NCORE * N_PAD, HALF), jnp.float32),
    mesh=_MESH,
    compiler_params=_PARAMS,
    scratch_types=[
        pltpu.VMEM((CHUNK,), jnp.int32),
        pltpu.VMEM((CHUNK,), jnp.int32),
        pltpu.VMEM((K, SUB), jnp.int32),
        pltpu.VMEM((K, SUB), jnp.int32),
        pltpu.VMEM((CHUNK,), jnp.float32),
        pltpu.VMEM((CHUNK,), jnp.float32),
        pltpu.VMEM((CHUNK, HALF), jnp.float32),
        pltpu.VMEM((CHUNK, HALF), jnp.float32),
        pltpu.VMEM_SHARED((N_PAD, HALF), jnp.float32),
        pltpu.SemaphoreType.DMA,
        pltpu.SemaphoreType.DMA,
        pltpu.SemaphoreType.DMA,
        pltpu.SemaphoreType.DMA,
        pltpu.SemaphoreType.DMA,
        pltpu.SemaphoreType.DMA,
        pltpu.SemaphoreType.DMA,
    ],
)(_spmm_body)


def _final_body(e0, e1, e2, e3, nid_both, out,
                nbuf, g0, g1, g2, g3, ob, sem):
    c = lax.axis_index("c")
    tid = lax.axis_index("s")
    pltpu.async_copy(nid_both.at[c * NSUB + tid], nbuf, sem).wait()

    descs = []
    for tbl, g in ((e0, g0), (e1, g1), (e2, g2), (e3, g3)):
        for q in range(BPT // SUB):
            descs.append(
                pltpu.async_copy(tbl.at[nbuf.at[pl.ds(q * SUB, SUB)]],
                                 g.at[pl.ds(q * SUB, SUB)], sem))
    for d in descs:
        d.wait()

    def mean(e, carry):
        ob[e, :] = (g0[e, :] + g1[e, :] + g2[e, :] + g3[e, :]) * 0.25
        return carry
    lax.fori_loop(0, BPT, mean, None, unroll=8)

    pltpu.sync_copy(ob, out.at[pl.ds(c * B2 + tid * BPT, BPT)])


_final = functools.partial(
    pl.kernel,
    out_type=jax.ShapeDtypeStruct((NCORE * B2, HALF), jnp.float32),
    mesh=_MESH,
    compiler_params=_PARAMS,
    scratch_types=[
        pltpu.VMEM((BPT,), jnp.int32),
        pltpu.VMEM((BPT, HALF), jnp.float32),
        pltpu.VMEM((BPT, HALF), jnp.float32),
        pltpu.VMEM((BPT, HALF), jnp.float32),
        pltpu.VMEM((BPT, HALF), jnp.float32),
        pltpu.VMEM((BPT, HALF), jnp.float32),
        pltpu.SemaphoreType.DMA,
    ],
)(_final_body)


def kernel(users, items, user_emb, item_emb, adj_rows, adj_cols, adj_vals):
    ego0 = jnp.concatenate([user_emb, item_emb], axis=0)
    ego0 = jnp.pad(ego0, ((0, N_PAD - N), (0, 0)))
    ego0 = ego0.reshape(N_PAD, NCORE, HALF).transpose(1, 0, 2).reshape(NCORE * N_PAD, HALF)
    cols_both = jnp.concatenate(
        [adj_cols, adj_cols + N_PAD]).reshape(2 * NCH, CHUNK)
    rows3 = adj_rows.reshape(NCH, K, SUB)
    vals2 = adj_vals.reshape(NCH, CHUNK)

    e1 = _spmm(ego0, cols_both, rows3, vals2)
    e2 = _spmm(e1, cols_both, rows3, vals2)
    e3 = _spmm(e2, cols_both, rows3, vals2)

    nid = jnp.concatenate(
        [users.astype(jnp.int32), items.astype(jnp.int32) + USERS])
    nid_both = jnp.concatenate([nid, nid + N_PAD]).reshape(2 * NSUB, BPT)
    outf = _final(ego0, e1, e2, e3, nid_both)

    o = outf.reshape(NCORE, B2, HALF).transpose(1, 0, 2).reshape(B2, EMB)
    return (o[:BATCH], o[BATCH:])


# speedup vs baseline: 23.7621x; 1.2303x over previous
import functools

import jax
import jax.numpy as jnp
from jax import lax
from jax.experimental import pallas as pl
from jax.experimental.pallas import tpu as pltpu
from jax.experimental.pallas import tpu_sc as plsc

USERS = 50000
ITEMS = 50000
N = 100000
N_PAD = 100096
EMB = 32
HALF = 16
E = 1600000
SUB = 128
K = 5
CHUNK = K * SUB
NCH = E // CHUNK
NCORE = 2
NSUB = 16
TRIPS = -(-NCH // NSUB)
T2 = (TRIPS + 2) // 2
ROWS_PER_TILE = N_PAD // NSUB
BATCH = 4096
B2 = 2 * BATCH
BPT = B2 // NSUB

_PARAMS = pltpu.CompilerParams(use_tc_tiling_on_sc=False, needs_layout_passes=False)

_MESH = plsc.VectorSubcoreMesh(
    core_axis_name="c", subcore_axis_name="s", num_cores=NCORE,
    num_subcores=NSUB)


def _spmm_body(ego, cols3, rv3, out,
               cbA, cbB, rvA, rvB, gA, gB, acc,
               sem_cbA, sem_cbB, sem_rvA, sem_rvB,
               sem_gA, sem_gB, sem_sA, sem_sB, sem_z):
    c = lax.axis_index("c")
    tid = lax.axis_index("s")

    bufA = (cbA, rvA, gA, sem_cbA, sem_rvA, sem_gA, sem_sA)
    bufB = (cbB, rvB, gB, sem_cbB, sem_rvB, sem_gB, sem_sB)

    def fire_cb(i, buf):
        pltpu.async_copy(cols3.at[c * NCH + i], buf[0], buf[3])

    def wait_cb(buf):
        pltpu.make_async_copy(cols3.at[c * NCH], buf[0], buf[3]).wait()

    def fire_rv(i, buf):
        pltpu.async_copy(rv3.at[i], buf[1], buf[4])

    def wait_rv(buf):
        pltpu.make_async_copy(rv3.at[0], buf[1], buf[4]).wait()

    def fire_gathers(buf):
        cb, g, sem_g = buf[0], buf[2], buf[5]
        for j in range(K):
            pltpu.async_copy(ego.at[cb.at[j]], g.at[pl.ds(j * SUB, SUB)],
                             sem_g)

    def wait_gathers(buf):
        cb, g, sem_g = buf[0], buf[2], buf[5]
        for j in range(K):
            pltpu.make_async_copy(ego.at[cb.at[j]],
                                  g.at[pl.ds(j * SUB, SUB)], sem_g).wait()

    def fire_scatters(buf):
        rv, g, sem_s = buf[1], buf[2], buf[6]
        for j in range(K):
            pltpu.async_copy(g.at[pl.ds(j * SUB, SUB)],
                             acc.at[rv.at[j]], sem_s, add=True)

    def wait_scatters(buf):
        rv, g, sem_s = buf[1], buf[2], buf[6]
        for j in range(K):
            pltpu.make_async_copy(g.at[pl.ds(j * SUB, SUB)],
                                  acc.at[rv.at[j]], sem_s).wait()

    def scale(buf):
        rv, g = buf[1], buf[2]

        def body(gi, carry):
            vi = rv[K + gi // 8, pl.ds((gi % 8) * 16, 16)]
            v = plsc.bitcast(vi, jnp.float32)
            base = gi * 16
            for l in range(16):
                g[base + l, :] = g[base + l, :] * v[l]
            return carry
        lax.fori_loop(0, CHUNK // 16, body, None, unroll=2)

    fire_cb(tid, bufA)
    fire_rv(tid, bufA)
    fire_cb(NSUB + tid, bufB)
    wait_cb(bufA)
    fire_gathers(bufA)

    def zbody(i, carry):
        gB[i, :] = jnp.zeros((HALF,), jnp.float32)
        return carry
    lax.fori_loop(0, CHUNK, zbody, None, unroll=8)
    row0 = tid * ROWS_PER_TILE
    zds = []
    for q in range(ROWS_PER_TILE // CHUNK):
        zds.append(pltpu.async_copy(
            gB, acc.at[pl.ds(row0 + q * CHUNK, CHUNK)], sem_z))
    tail = ROWS_PER_TILE % CHUNK
    if tail:
        zds.append(pltpu.async_copy(
            gB.at[pl.ds(0, tail)],
            acc.at[pl.ds(row0 + ROWS_PER_TILE - tail, tail)], sem_z))
    for d in zds:
        d.wait()
    plsc.subcore_barrier()

    def half(t, cur, nxt):
        i_cur = t * NSUB + tid
        i_prev = i_cur - NSUB
        i_next = i_cur + NSUB
        i_next2 = i_cur + 2 * NSUB

        @pl.when(i_cur < NCH)
        def _():
            wait_gathers(cur)

        @pl.when((t >= 1) & (i_prev < NCH))
        def _():
            wait_scatters(nxt)

        @pl.when(i_next < NCH)
        def _():
            fire_rv(i_next, nxt)

        @pl.when(i_next2 < NCH)
        def _():
            fire_cb(i_next2, cur)

        @pl.when(i_next < NCH)
        def _():
            wait_cb(nxt)
            fire_gathers(nxt)

        @pl.when(i_cur < NCH)
        def _():
            wait_rv(cur)
            scale(cur)
            fire_scatters(cur)

    def pipe(t2, carry):
        half(2 * t2, bufA, bufB)
        half(2 * t2 + 1, bufB, bufA)
        return carry
    lax.fori_loop(0, T2, pipe, None)

    plsc.subcore_barrier()
    pltpu.sync_copy(acc.at[pl.ds(row0, ROWS_PER_TILE)],
                    out.at[pl.ds(c * N_PAD + row0, ROWS_PER_TILE)])


_spmm = functools.partial(
    pl.kernel,
    out_type=jax.ShapeDtypeStruct((NCORE * N_PAD, HALF), jnp.float32),
    mesh=_MESH,
    compiler_params=_PARAMS,
    scratch_types=[
        pltpu.VMEM((K, SUB), jnp.int32),
        pltpu.VMEM((K, SUB), jnp.int32),
        pltpu.VMEM((2 * K, SUB), jnp.int32),
        pltpu.VMEM((2 * K, SUB), jnp.int32),
        pltpu.VMEM((CHUNK, HALF), jnp.float32),
        pltpu.VMEM((CHUNK, HALF), jnp.float32),
        pltpu.VMEM_SHARED((N_PAD, HALF), jnp.float32),
        pltpu.SemaphoreType.DMA,
        pltpu.SemaphoreType.DMA,
        pltpu.SemaphoreType.DMA,
        pltpu.SemaphoreType.DMA,
        pltpu.SemaphoreType.DMA,
        pltpu.SemaphoreType.DMA,
        pltpu.SemaphoreType.DMA,
        pltpu.SemaphoreType.DMA,
        pltpu.SemaphoreType.DMA,
    ],
)(_spmm_body)


def _final_body(e0, e1, e2, e3, nid_both, out,
                nbuf, g0, g1, g2, g3, ob, sem):
    c = lax.axis_index("c")
    tid = lax.axis_index("s")
    pltpu.async_copy(nid_both.at[c * NSUB + tid], nbuf, sem).wait()

    descs = []
    for tbl, g in ((e0, g0), (e1, g1), (e2, g2), (e3, g3)):
        for q in range(BPT // SUB):
            descs.append(
                pltpu.async_copy(tbl.at[nbuf.at[pl.ds(q * SUB, SUB)]],
                                 g.at[pl.ds(q * SUB, SUB)], sem))
    for d in descs:
        d.wait()

    def mean(e, carry):
        ob[e, :] = (g0[e, :] + g1[e, :] + g2[e, :] + g3[e, :]) * 0.25
        return carry
    lax.fori_loop(0, BPT, mean, None, unroll=8)

    pltpu.sync_copy(ob, out.at[pl.ds(c * B2 + tid * BPT, BPT)])


_final = functools.partial(
    pl.kernel,
    out_type=jax.ShapeDtypeStruct((NCORE * B2, HALF), jnp.float32),
    mesh=_MESH,
    compiler_params=_PARAMS,
    scratch_types=[
        pltpu.VMEM((BPT,), jnp.int32),
        pltpu.VMEM((BPT, HALF), jnp.float32),
        pltpu.VMEM((BPT, HALF), jnp.float32),
        pltpu.VMEM((BPT, HALF), jnp.float32),
        pltpu.VMEM((BPT, HALF), jnp.float32),
        pltpu.VMEM((BPT, HALF), jnp.float32),
        pltpu.SemaphoreType.DMA,
    ],
)(_final_body)


def kernel(users, items, user_emb, item_emb, adj_rows, adj_cols, adj_vals):
    ego0 = jnp.concatenate([user_emb, item_emb], axis=0)
    ego0 = jnp.pad(ego0, ((0, N_PAD - N), (0, 0)))
    ego0 = ego0.reshape(N_PAD, NCORE, HALF).transpose(1, 0, 2).reshape(NCORE * N_PAD, HALF)
    cols3 = jnp.concatenate(
        [adj_cols, adj_cols + N_PAD]).reshape(2 * NCH, K, SUB)
    rv3 = jnp.concatenate(
        [adj_rows.reshape(NCH, K, SUB),
         jax.lax.bitcast_convert_type(adj_vals, jnp.int32).reshape(NCH, K, SUB)],
        axis=1)

    e1 = _spmm(ego0, cols3, rv3)
    e2 = _spmm(e1, cols3, rv3)
    e3 = _spmm(e2, cols3, rv3)

    nid = jnp.concatenate(
        [users.astype(jnp.int32), items.astype(jnp.int32) + USERS])
    nid_both = jnp.concatenate([nid, nid + N_PAD]).reshape(2 * NSUB, BPT)
    outf = _final(ego0, e1, e2, e3, nid_both)

    o = outf.reshape(NCORE, B2, HALF).transpose(1, 0, 2).reshape(B2, EMB)
    return (o[:BATCH], o[BATCH:])


# speedup vs baseline: 26.4883x; 1.1147x over previous
import functools

import jax
import jax.numpy as jnp
from jax import lax
from jax.experimental import pallas as pl
from jax.experimental.pallas import tpu as pltpu
from jax.experimental.pallas import tpu_sc as plsc

USERS = 50000
ITEMS = 50000
N = 100000
N_PAD = 100096
EMB = 32
HALF = 16
E = 1600000
SUB = 128
K = 6
CHUNK = K * SUB
NCH = -(-E // CHUNK)
E_PAD = NCH * CHUNK
NCORE = 2
NSUB = 16
TRIPS = -(-NCH // NSUB)
T2 = (TRIPS + 2) // 2
ROWS_PER_TILE = N_PAD // NSUB
BATCH = 4096
B2 = 2 * BATCH
BPT = B2 // NSUB
FSUB = 256

_PARAMS = pltpu.CompilerParams(use_tc_tiling_on_sc=False,
                               needs_layout_passes=False)

_MESH = plsc.VectorSubcoreMesh(
    core_axis_name="c", subcore_axis_name="s", num_cores=NCORE,
    num_subcores=NSUB)


def _gcn_body(ego0, cols3, rv3, nid3, out, t1, t2, t3,
              cbA, cbB, rvA, rvB, gA, gB, acc,
              sem_cbA, sem_cbB, sem_rvA, sem_rvB,
              sem_gA, sem_gB, sem_sA, sem_sB, sem_z):
    c = lax.axis_index("c")
    tid = lax.axis_index("s")
    row0 = tid * ROWS_PER_TILE

    bufA = (cbA, rvA, gA, sem_cbA, sem_rvA, sem_gA, sem_sA)
    bufB = (cbB, rvB, gB, sem_cbB, sem_rvB, sem_gB, sem_sB)

    def fire_cb(i, buf):
        pltpu.async_copy(cols3.at[i], buf[0], buf[3])

    def wait_cb(buf):
        pltpu.make_async_copy(cols3.at[0], buf[0], buf[3]).wait()

    def fire_rv(i, buf):
        pltpu.async_copy(rv3.at[i], buf[1], buf[4])

    def wait_rv(buf):
        pltpu.make_async_copy(rv3.at[0], buf[1], buf[4]).wait()

    def spmm_phase(src, dst):
        def fire_gathers(buf):
            cb, g, sem_g = buf[0], buf[2], buf[5]
            for j in range(K):
                pltpu.async_copy(src.at[c].at[cb.at[j]],
                                 g.at[pl.ds(j * SUB, SUB)], sem_g)

        def wait_gathers(buf):
            cb, g, sem_g = buf[0], buf[2], buf[5]
            for j in range(K):
                pltpu.make_async_copy(src.at[c].at[cb.at[j]],
                                      g.at[pl.ds(j * SUB, SUB)], sem_g).wait()

        def fire_scatters(buf):
            rv, g, sem_s = buf[1], buf[2], buf[6]
            for j in range(K):
                pltpu.async_copy(g.at[pl.ds(j * SUB, SUB)],
                                 acc.at[rv.at[j]], sem_s, add=True)

        def wait_scatters(buf):
            rv, g, sem_s = buf[1], buf[2], buf[6]
            for j in range(K):
                pltpu.make_async_copy(g.at[pl.ds(j * SUB, SUB)],
                                      acc.at[rv.at[j]], sem_s).wait()

        def scale(buf):
            rv, g = buf[1], buf[2]

            def body(gi, carry):
                vi = rv[K + gi // 8, pl.ds((gi % 8) * 16, 16)]
                v = plsc.bitcast(vi, jnp.float32)
                base = gi * 16
                for l in range(16):
                    g[base + l, :] = g[base + l, :] * v[l]
                return carry
            lax.fori_loop(0, CHUNK // 16, body, None, unroll=2)

        fire_cb(tid, bufA)
        fire_rv(tid, bufA)
        fire_cb(NSUB + tid, bufB)
        wait_cb(bufA)
        fire_gathers(bufA)

        def zbody(i, carry):
            gB[i, :] = jnp.zeros((HALF,), jnp.float32)
            return carry
        lax.fori_loop(0, CHUNK, zbody, None, unroll=8)
        zds = []
        for q in range(ROWS_PER_TILE // CHUNK):
            zds.append(pltpu.async_copy(
                gB, acc.at[pl.ds(row0 + q * CHUNK, CHUNK)], sem_z))
        tail = ROWS_PER_TILE % CHUNK
        if tail:
            zds.append(pltpu.async_copy(
                gB.at[pl.ds(0, tail)],
                acc.at[pl.ds(row0 + ROWS_PER_TILE - tail, tail)], sem_z))
        for d in zds:
            d.wait()
        plsc.subcore_barrier()

        def half(t, cur, nxt):
            i_cur = t * NSUB + tid
            i_prev = i_cur - NSUB
            i_next = i_cur + NSUB
            i_next2 = i_cur + 2 * NSUB

            @pl.when(i_cur < NCH)
            def _():
                wait_gathers(cur)

            @pl.when((t >= 1) & (i_prev < NCH))
            def _():
                wait_scatters(nxt)

            @pl.when(i_next < NCH)
            def _():
                fire_rv(i_next, nxt)

            @pl.when(i_next2 < NCH)
            def _():
                fire_cb(i_next2, cur)

            @pl.when(i_next < NCH)
            def _():
                wait_cb(nxt)
                fire_gathers(nxt)

            @pl.when(i_cur < NCH)
            def _():
                wait_rv(cur)
                scale(cur)
                fire_scatters(cur)

        def pipe(t2_, carry):
            half(2 * t2_, bufA, bufB)
            half(2 * t2_ + 1, bufB, bufA)
            return carry
        lax.fori_loop(0, T2, pipe, None)

        plsc.subcore_barrier()
        pltpu.sync_copy(acc.at[pl.ds(row0, ROWS_PER_TILE)],
                        dst.at[c].at[pl.ds(row0, ROWS_PER_TILE)])
        plsc.subcore_barrier()

    spmm_phase(ego0, t1)
    spmm_phase(t1, t2)
    spmm_phase(t2, t3)

    for p in range(2):
        pltpu.async_copy(nid3.at[tid * 2 + p], cbA.at[pl.ds(0, 2)],
                         sem_cbA).wait()
        descs = []
        for li, tbl in enumerate((ego0, t1, t2, t3)):
            for q in range(2):
                dgbuf = gA if li < 3 else gB
                doff = li * FSUB if li < 3 else 0
                descs.append(pltpu.async_copy(
                    tbl.at[c].at[cbA.at[q]],
                    dgbuf.at[pl.ds(doff + q * SUB, SUB)], sem_gA))
        for d in descs:
            d.wait()

        def mean(e, carry):
            m = (gA[e, :] + gA[FSUB + e, :] + gA[2 * FSUB + e, :]
                 + gB[e, :]) * 0.25
            gB[FSUB + e, :] = m
            return carry
        lax.fori_loop(0, FSUB, mean, None, unroll=8)

        pltpu.sync_copy(
            gB.at[pl.ds(FSUB, FSUB)],
            out.at[c].at[pl.ds(tid * BPT + p * FSUB, FSUB)])


_gcn = functools.partial(
    pl.kernel,
    out_type=jax.ShapeDtypeStruct((NCORE, B2, HALF), jnp.float32),
    mesh=_MESH,
    compiler_params=_PARAMS,
    scratch_types=[
        pltpu.HBM((NCORE, N_PAD, HALF), jnp.float32),
        pltpu.HBM((NCORE, N_PAD, HALF), jnp.float32),
        pltpu.HBM((NCORE, N_PAD, HALF), jnp.float32),
        pltpu.VMEM((K, SUB), jnp.int32),
        pltpu.VMEM((K, SUB), jnp.int32),
        pltpu.VMEM((2 * K, SUB), jnp.int32),
        pltpu.VMEM((2 * K, SUB), jnp.int32),
        pltpu.VMEM((CHUNK, HALF), jnp.float32),
        pltpu.VMEM((CHUNK, HALF), jnp.float32),
        pltpu.VMEM_SHARED((N_PAD, HALF), jnp.float32),
        pltpu.SemaphoreType.DMA,
        pltpu.SemaphoreType.DMA,
        pltpu.SemaphoreType.DMA,
        pltpu.SemaphoreType.DMA,
        pltpu.SemaphoreType.DMA,
        pltpu.SemaphoreType.DMA,
        pltpu.SemaphoreType.DMA,
        pltpu.SemaphoreType.DMA,
        pltpu.SemaphoreType.DMA,
    ],
)(_gcn_body)


def kernel(users, items, user_emb, item_emb, adj_rows, adj_cols, adj_vals):
    ego0 = jnp.concatenate([user_emb, item_emb], axis=0)
    ego0 = jnp.pad(ego0, ((0, N_PAD - N), (0, 0)))
    ego0 = ego0.reshape(N_PAD, NCORE, HALF).transpose(1, 0, 2)

    cols_p = jnp.pad(adj_cols, (0, E_PAD - E))
    rows_p = jnp.pad(adj_rows, (0, E_PAD - E))
    vals_p = jnp.pad(adj_vals, (0, E_PAD - E))
    cols3 = cols_p.reshape(NCH, K, SUB)
    rv3 = jnp.concatenate(
        [rows_p.reshape(NCH, K, SUB),
         jax.lax.bitcast_convert_type(vals_p, jnp.int32).reshape(NCH, K, SUB)],
        axis=1)

    nid = jnp.concatenate(
        [users.astype(jnp.int32), items.astype(jnp.int32) + USERS])
    nid3 = nid.reshape(NSUB * 2, 2, SUB)

    outf = _gcn(ego0, cols3, rv3, nid3)

    o = outf.transpose(1, 0, 2).reshape(B2, EMB)
    return (o[:BATCH], o[BATCH:])
